# baseline (speedup 1.0000x reference)
import functools

import jax
import jax.numpy as jnp
from jax import lax
from jax.experimental import pallas as pl
from jax.experimental.pallas import tpu as pltpu
from jax.experimental.pallas import tpu_sc as plsc

N_NODES = 10000
N_EDGES = 320000
D_IN = 128
D_HID = 128
D_OUT = 64
DQ = 32
NQ = 4

NC = 2
NS = 16
NW = NC * NS
NP = 10240
RPT = NP // NS
K = 128
NCHT = N_EDGES // K
NCH_LO = NCHT // NW
NREM = NCHT - NCH_LO * NW
NCH_HI = NCH_LO + 1
NCH_UP = NCH_LO + 2

_MESH = plsc.VectorSubcoreMesh(core_axis_name="c", subcore_axis_name="s")
_SC_PARAMS = pltpu.CompilerParams(use_tc_tiling_on_sc=False)


def _chunks(c, s):
  wid = s * NC + c
  base = wid * NCH_LO + jnp.minimum(wid, NREM)
  nch = jnp.where(wid < NREM, NCH_HI, NCH_LO)
  start = base * K
  start_dma = jnp.minimum(start, N_EDGES - NCH_UP * K)
  return start_dma, start - start_dma, nch


def _deg_body(edge_hbm, zero_hbm, out_hbm, idx_d, ones_v, degacc, isem):
  c = lax.axis_index("c")
  s = lax.axis_index("s")
  start_dma, off, nch = _chunks(c, s)
  cp = pltpu.async_copy(edge_hbm.at[1].at[pl.ds(start_dma, NCH_UP * K)], idx_d, isem)
  for i in range(K // 16):
    ones_v[pl.ds(i * 16, 16)] = jnp.full((16,), 1.0, jnp.float32)
  pltpu.sync_copy(zero_hbm.at[pl.ds(s * RPT, RPT)], degacc.at[pl.ds(s * RPT, RPT)])
  cp.wait()
  plsc.subcore_barrier()

  @pl.loop(0, NCH_UP)
  def _(j):
    @pl.when(j < nch)
    def _():
      pltpu.sync_copy(ones_v, degacc.at[idx_d.at[pl.ds(off + j * K, K)]], add=True)

  plsc.subcore_barrier()
  pltpu.sync_copy(degacc.at[pl.ds(s * RPT, RPT)], out_hbm.at[c].at[pl.ds(s * RPT, RPT)])


@functools.partial(
    pl.kernel,
    out_type=jax.ShapeDtypeStruct((NC, NP), jnp.float32),
    mesh=_MESH,
    scratch_types=[
        pltpu.VMEM((NCH_UP * K,), jnp.int32),
        pltpu.VMEM((K,), jnp.float32),
        pltpu.VMEM_SHARED((NP,), jnp.float32),
        pltpu.SemaphoreType.DMA,
    ],
    compiler_params=_SC_PARAMS,
)
def _deg_kernel(edge_hbm, zero_hbm, out_hbm, idx_d, ones_v, degacc, isem):
  _deg_body(edge_hbm, zero_hbm, out_hbm, idx_d, ones_v, degacc, isem)


_ND = 6
_NG = 4


def _spmm_body(edge_hbm, p4_hbm, out_hbm, refs):
  idx_s, idx_d = refs[0], refs[1]
  rows = refs[2:2 + _ND]
  zbuf, pq0, pq1, acc = refs[2 + _ND:6 + _ND]
  isem0, isem1 = refs[6 + _ND], refs[7 + _ND]
  gsem = refs[8 + _ND:8 + 2 * _ND]
  csem = refs[8 + 2 * _ND:8 + 3 * _ND]
  ssem0, ssem1 = refs[8 + 3 * _ND], refs[9 + 3 * _ND]
  c = lax.axis_index("c")
  s = lax.axis_index("s")
  start_dma, off, nch = _chunks(c, s)
  cps = pltpu.async_copy(edge_hbm.at[0].at[pl.ds(start_dma, NCH_UP * K)], idx_s, isem0)
  cpd = pltpu.async_copy(edge_hbm.at[1].at[pl.ds(start_dma, NCH_UP * K)], idx_d, isem1)

  @pl.loop(0, RPT)
  def _(r):
    for cc in range(DQ // 16):
      zbuf[r, pl.ds(cc * 16, 16)] = jnp.zeros((16,), jnp.float32)

  pqs = (pq0, pq1)
  ssem = (ssem0, ssem1)
  rslice = pl.ds(s * RPT, RPT)

  def stage(q, sync):
    cp = pltpu.async_copy(p4_hbm.at[rslice, pl.ds(q * DQ, DQ)],
                          pqs[q % 2].at[rslice], ssem[q % 2])
    if sync:
      cp.wait()

  stage(0, True)
  stage(1, False)
  pltpu.sync_copy(zbuf, acc.at[rslice])
  cps.wait()
  cpd.wait()
  plsc.subcore_barrier()

  def wait_gather(b):
    pltpu.make_async_copy(pqs[0].at[idx_s.at[pl.ds(0, K)]], rows[b],
                          gsem[b]).wait()

  def wait_scatter(b):
    pltpu.make_async_copy(rows[b], acc.at[idx_d.at[pl.ds(0, K)]],
                          csem[b]).wait()

  for q in range(NQ):
    pq = pqs[q % 2]
    for b in range(_NG):
      pltpu.async_copy(pq.at[idx_s.at[pl.ds(off + b * K, K)]], rows[b], gsem[b])

    @pl.loop(0, NCH_UP, step=_ND)
    def _(jj):
      for b in range(_ND):
        j = jj + b

        @pl.when(j < nch)
        def _():
          wait_gather(b)
          pltpu.async_copy(rows[b], acc.at[idx_d.at[pl.ds(off + j * K, K)]],
                           csem[b], add=True)

        @pl.when(j + _NG < nch)
        def _():
          @pl.when(j >= _ND - _NG)
          def _():
            wait_scatter((b + _NG) % _ND)

          pltpu.async_copy(pq.at[idx_s.at[pl.ds(off + (j + _NG) * K, K)]],
                           rows[(b + _NG) % _ND], gsem[(b + _NG) % _ND])

    for b in range(_ND):
      wait_scatter(b)

    plsc.subcore_barrier()
    pltpu.sync_copy(acc.at[rslice], out_hbm.at[c].at[rslice, pl.ds(q * DQ, DQ)])
    if q + 1 < NQ:
      pltpu.sync_copy(zbuf, acc.at[rslice])
      if q + 2 < NQ:
        stage(q + 2, False)
      pltpu.make_async_copy(p4_hbm.at[rslice, pl.ds((q + 1) * DQ, DQ)],
                            pqs[(q + 1) % 2].at[rslice], ssem[(q + 1) % 2]).wait()
      plsc.subcore_barrier()


@functools.partial(
    pl.kernel,
    out_type=jax.ShapeDtypeStruct((NC, NP, NQ * DQ), jnp.float32),
    mesh=_MESH,
    scratch_types=[
        pltpu.VMEM((NCH_UP * K,), jnp.int32),
        pltpu.VMEM((NCH_UP * K,), jnp.int32),
    ] + [pltpu.VMEM((K, DQ), jnp.float32)] * _ND + [
        pltpu.VMEM((RPT, DQ), jnp.float32),
        pltpu.VMEM_SHARED((NP, DQ), jnp.float32),
        pltpu.VMEM_SHARED((NP, DQ), jnp.float32),
        pltpu.VMEM_SHARED((NP, DQ), jnp.float32),
    ] + [pltpu.SemaphoreType.DMA] * (4 + 2 * _ND),
    compiler_params=_SC_PARAMS,
)
def _spmm_kernel(edge_hbm, p4_hbm, out_hbm, *refs):
  _spmm_body(edge_hbm, p4_hbm, out_hbm, refs)


_BR = 1280
_GRID = NP // _BR
_BR3 = 1000
_GRID3 = N_NODES // _BR3


def _rsqrt_deg(d_ref):
  deg = d_ref[:, 0:1] + d_ref[:, 1:2] + 1.0
  return lax.rsqrt(deg)


def _tc1a_body(x_ref, w_ref, o_ref):
  o_ref[...] = jnp.dot(x_ref[...], w_ref[...],
                       preferred_element_type=jnp.float32)


def _tc1b_body(u_ref, d_ref, o_ref):
  o_ref[...] = _rsqrt_deg(d_ref) * u_ref[...]


def _spmm_sum(s_ref, p_ref):
  return s_ref[0] + s_ref[1] + p_ref[...]


def _tc2_body(s_ref, p_ref, d_ref, b_ref, w_ref, o_ref):
  q = _rsqrt_deg(d_ref)
  h = q * _spmm_sum(s_ref, p_ref) + b_ref[...]
  h = jnp.maximum(h, 0.0)
  o_ref[...] = q * jnp.dot(h, w_ref[...], preferred_element_type=jnp.float32)


def _tc3_body(s_ref, p_ref, d_ref, b2_ref, b3_ref, mu_ref, lv_ref):
  q = _rsqrt_deg(d_ref)
  t = q * _spmm_sum(s_ref, p_ref)
  mu_ref[...] = t[:, :D_OUT] + b2_ref[...]
  lv_ref[...] = t[:, D_OUT:] + b3_ref[...]


def _row_spec(width, br=_BR):
  return pl.BlockSpec((br, width), lambda i: (i, 0))


def _full_spec(shape):
  return pl.BlockSpec(shape, lambda i: (0,) * len(shape))


def _s_spec(br=_BR):
  return pl.BlockSpec((NC, br, NQ * DQ), lambda i: (0, i, 0))


_p_out = jax.ShapeDtypeStruct((NP, D_HID), jnp.float32)


def _tc1a(x_pad, W1):
  return pl.pallas_call(
      _tc1a_body,
      grid=(_GRID,),
      in_specs=[_row_spec(128), _full_spec((128, D_HID))],
      out_specs=_row_spec(D_HID),
      out_shape=_p_out,
  )(x_pad, W1)


def _tc1b(u, degT):
  return pl.pallas_call(
      _tc1b_body,
      grid=(_GRID,),
      in_specs=[_row_spec(128), _row_spec(2)],
      out_specs=_row_spec(D_HID),
      out_shape=_p_out,
  )(u, degT)


def _tc2(s1, p, degT, b1, Wc):
  return pl.pallas_call(
      _tc2_body,
      grid=(_GRID,),
      in_specs=[_s_spec(), _row_spec(128), _row_spec(2),
                _full_spec((1, 128)), _full_spec((128, 128))],
      out_specs=_row_spec(D_HID),
      out_shape=_p_out,
  )(s1, p, degT, b1, Wc)


def _tc3(s2, q, degT, b2, b3):
  return pl.pallas_call(
      _tc3_body,
      grid=(_GRID3,),
      in_specs=[_s_spec(_BR3), _row_spec(128, _BR3), _row_spec(2, _BR3),
                _full_spec((1, D_OUT)), _full_spec((1, D_OUT))],
      out_specs=[_row_spec(D_OUT, _BR3), _row_spec(D_OUT, _BR3)],
      out_shape=[jax.ShapeDtypeStruct((N_NODES, D_OUT), jnp.float32)] * 2,
  )(s2, q, degT, b2, b3)


def kernel(x, edge_index, W1, b1, W2, b2, W3, b3):
  ei = edge_index.astype(jnp.int32)
  x_pad = jnp.pad(x, ((0, NP - N_NODES), (0, 0)))
  zero1 = jnp.zeros((NP,), jnp.float32)
  b1r = b1.reshape(1, D_HID)
  b2r = b2.reshape(1, D_OUT)
  b3r = b3.reshape(1, D_OUT)
  Wc = jnp.concatenate([W2, W3], axis=1)

  deg2 = _deg_kernel(ei, zero1)
  degT = deg2.T
  u = _tc1a(x_pad, W1)
  p4 = _tc1b(u, degT)
  s1 = _spmm_kernel(ei, p4)
  q4 = _tc2(s1, p4, degT, b1r, Wc)
  s2 = _spmm_kernel(ei, q4)
  return _tc3(s2, q4, degT, b2r, b3r)
